# baseline (speedup 1.0000x reference)
import functools

import jax
import jax.numpy as jnp
from jax import lax
from jax.experimental import pallas as pl
from jax.experimental.pallas import tpu as pltpu
from jax.experimental.pallas import tpu_sc as plsc

N_STATES = 1000000
OUTPUT_DIM = 16
BATCH = 16384

_info = plsc.get_sparse_core_info()
_NC, _NS, _NL = _info.num_cores, _info.num_subcores, _info.num_lanes
_NW = _NC * _NS
_B_PER_W = BATCH // _NW
_E_PER_W = _B_PER_W * OUTPUT_DIM
_CHUNK = 128
_NCHUNK = _E_PER_W // _CHUNK

_mesh = plsc.VectorSubcoreMesh(core_axis_name="c", subcore_axis_name="s")


@functools.partial(
    pl.kernel,
    mesh=_mesh,
    out_type=jax.ShapeDtypeStruct((BATCH * OUTPUT_DIM,), jnp.float32),
    scratch_types=[
        pltpu.VMEM((_B_PER_W,), jnp.int32),
        pltpu.VMEM((_E_PER_W,), jnp.int32),
        pltpu.VMEM((_E_PER_W,), jnp.float32),
        pltpu.SemaphoreType.DMA,
    ],
    compiler_params=pltpu.CompilerParams(needs_layout_passes=False),
)
def _gather_kernel(table_hbm, idx_hbm, out_hbm, idx_v, eidx_v, rows_v, sem):
    wid = lax.axis_index("s") * _NC + lax.axis_index("c")
    base = wid * _B_PER_W
    pltpu.sync_copy(idx_hbm.at[pl.ds(base, _B_PER_W)], idx_v)

    iota = lax.iota(jnp.int32, _NL)

    def _expand(i, _):
        for b in range(4):
            j = i * 4 + b
            row = plsc.load_gather(idx_v, [jnp.full((_NL,), j, jnp.int32)])
            eidx_v[pl.ds(j * _NL, _NL)] = row * OUTPUT_DIM + iota
        return ()

    lax.fori_loop(0, _B_PER_W // 4, _expand, ())

    def _fire(c, _):
        for b in range(4):
            j = c * 4 + b
            pltpu.make_async_copy(
                table_hbm.at[eidx_v.at[pl.ds(j * _CHUNK, _CHUNK)]],
                rows_v.at[pl.ds(j * _CHUNK, _CHUNK)],
                sem,
            ).start()
        return ()

    lax.fori_loop(0, _NCHUNK // 4, _fire, ())

    pltpu.make_async_copy(
        table_hbm.at[pl.ds(0, _E_PER_W)], rows_v, sem
    ).wait()

    pltpu.sync_copy(rows_v, out_hbm.at[pl.ds(base * OUTPUT_DIM, _E_PER_W)])


def kernel(preprocessed_states, table):
    idx = jnp.reshape(preprocessed_states, (BATCH,)).astype(jnp.int32)
    table1d = jnp.reshape(table, (N_STATES * OUTPUT_DIM,))
    out1d = _gather_kernel(table1d, idx)
    return jnp.reshape(out1d, (BATCH, OUTPUT_DIM))
